# baseline (speedup 1.0000x reference)
import functools

import jax
import jax.numpy as jnp
from jax import lax
from jax.experimental import pallas as pl
from jax.experimental.pallas import tpu as pltpu
from jax.experimental.pallas import tpu_sc as plsc

NC = 2
NS = 16
NW = NC * NS
LANES = 16
B = 80
CR = 25


def _make_counts_kernel(E, NP):
    EW = E // NW
    RW = EW // B
    NCH = RW // CR
    ST = NP // NS
    mesh = plsc.VectorSubcoreMesh(core_axis_name="c", subcore_axis_name="s")

    @functools.partial(
        pl.kernel,
        out_type=jax.ShapeDtypeStruct((NC * NP,), jnp.float32),
        mesh=mesh,
        compiler_params=pltpu.CompilerParams(needs_layout_passes=False),
        scratch_types=[
            pltpu.VMEM((CR, B), jnp.int32),
            pltpu.VMEM((B,), jnp.float32),
            pltpu.VMEM_SHARED((NP,), jnp.float32),
        ],
    )
    def counts_kernel(pair2_h, zeros_h, cnt_h, pairc, ones_v, cnt_sp):
        cid = lax.axis_index("c")
        sid = lax.axis_index("s")
        wid = sid * NC + cid
        for k in range(B // LANES):
            ones_v[pl.ds(k * LANES, LANES)] = jnp.ones((LANES,), jnp.float32)
        pltpu.sync_copy(zeros_h, cnt_sp.at[pl.ds(sid * ST, ST)])
        plsc.subcore_barrier()

        def chunk_body(c, carry):
            ch = wid * NCH + c
            pltpu.sync_copy(pair2_h.at[ch], pairc)

            def sb_body(sb, carry2):
                pltpu.sync_copy(ones_v, cnt_sp.at[pairc.at[sb]], add=True)
                return carry2

            return lax.fori_loop(0, CR, sb_body, carry)

        lax.fori_loop(0, NCH, chunk_body, 0)
        plsc.subcore_barrier()
        pltpu.sync_copy(cnt_sp.at[pl.ds(sid * ST, ST)],
                        cnt_h.at[pl.ds(cid * NP + sid * ST, ST)])

    return counts_kernel


def _make_agg_kernel(E, D, RS, first):
    EW = E // NW
    RW = EW // B
    NCH = RW // CR
    mesh = plsc.VectorSubcoreMesh(core_axis_name="c", subcore_axis_name="s")

    acc_t = jax.ShapeDtypeStruct((NC * NS, RS, D), jnp.float32)
    esc_t = jax.ShapeDtypeStruct((E // (CR * B), CR, B), jnp.float32)
    out_type = [acc_t, esc_t] if first else acc_t

    scratch = [
        pltpu.VMEM((CR, B), jnp.int32),
        pltpu.VMEM((CR, B), jnp.int32),
        pltpu.VMEM((CR, B), jnp.float32),
        pltpu.VMEM((B, D), jnp.float32),
        pltpu.VMEM((B, D), jnp.float32),
        pltpu.VMEM_SHARED((NS * RS, D), jnp.float32),
        pltpu.SemaphoreType.DMA,
        pltpu.SemaphoreType.DMA,
        pltpu.SemaphoreType.DMA,
        pltpu.SemaphoreType.DMA,
    ]
    if first:
        scratch.insert(2, pltpu.VMEM((CR, B), jnp.int32))

    if first:
        @functools.partial(
            pl.kernel, out_type=out_type, mesh=mesh,
            compiler_params=pltpu.CompilerParams(needs_layout_passes=False),
            scratch_types=scratch,
        )
        def agg_kernel(h_h, key2_h, dst2_h, pair2_h, norm_h, zrow_h, *rest):
            _agg_body(h_h, key2_h, dst2_h, pair2_h, norm_h, zrow_h, rest,
                      E, D, RS, True)
    else:
        @functools.partial(
            pl.kernel, out_type=out_type, mesh=mesh,
            compiler_params=pltpu.CompilerParams(needs_layout_passes=False),
            scratch_types=scratch,
        )
        def agg_kernel(h_h, key2_h, dst2_h, esc2_h, zrow_h, *rest):
            _agg_body(h_h, key2_h, dst2_h, esc2_h, None, zrow_h, rest,
                      E, D, RS, False)

    return agg_kernel


def _agg_body(h_h, key2_h, dst2_h, aux2_h, norm_h, zrow_h, rest,
              E, D, RS, first):
    EW = E // NW
    RW = EW // B
    NCH = RW // CR
    if first:
        (out_h, esc_h, keyc, dstc, pairc, sclc,
         rows0, rows1, acc_sp, semr0, semr1, sems0, sems1) = rest
    else:
        (out_h, keyc, dstc, sclc,
         rows0, rows1, acc_sp, semr0, semr1, sems0, sems1) = rest
        pairc = None
        esc_h = None
    cid = lax.axis_index("c")
    sid = lax.axis_index("s")
    wid = sid * NC + cid
    pltpu.sync_copy(zrow_h, acc_sp.at[pl.ds(sid * RS, RS)])
    plsc.subcore_barrier()

    bufs = ((rows0, semr0, sems0), (rows1, semr1, sems1))

    def issue(sb, p):
        rows, semr, sems = bufs[p]
        pltpu.async_copy(h_h.at[keyc.at[sb]], rows, semr)
        if first:
            pltpu.async_copy(norm_h.at[pairc.at[sb]], sclc.at[sb], sems)

    def drain(sb, p):
        rows, semr, sems = bufs[p]
        pltpu.make_async_copy(h_h.at[keyc.at[sb]], rows, semr).wait()
        if first:
            pltpu.make_async_copy(
                norm_h.at[pairc.at[sb]], sclc.at[sb], sems).wait()

    def scale_scatter(sb, p):
        rows, _, _ = bufs[p]

        def row_body(r, carry3):
            sc16 = plsc.load_gather(
                sclc.at[sb], [lax.broadcast(r, (LANES,))])
            for k in range(D // LANES):
                sl = pl.ds(k * LANES, LANES)
                rows[r, sl] = rows[r, sl] * sc16
            return carry3

        lax.fori_loop(0, B, row_body, 0)
        pltpu.sync_copy(rows, acc_sp.at[dstc.at[sb]], add=True)

    def chunk_body(c, carry):
        ch = wid * NCH + c
        pltpu.sync_copy(key2_h.at[ch], keyc)
        pltpu.sync_copy(dst2_h.at[ch], dstc)
        if first:
            pltpu.sync_copy(aux2_h.at[ch], pairc)
        else:
            pltpu.sync_copy(aux2_h.at[ch], sclc)

        issue(0, 0)

        def pair_body(i, carry2):
            sb = 2 * i
            drain(sb, 0)
            issue(sb + 1, 1)
            scale_scatter(sb, 0)
            drain(sb + 1, 1)
            issue(sb + 2, 0)
            scale_scatter(sb + 1, 1)
            return carry2

        lax.fori_loop(0, (CR - 1) // 2, pair_body, 0)
        drain(CR - 1, 0)
        scale_scatter(CR - 1, 0)
        if first:
            pltpu.sync_copy(sclc, esc_h.at[ch])
        return carry

    lax.fori_loop(0, NCH, chunk_body, 0)
    plsc.subcore_barrier()
    pltpu.sync_copy(acc_sp.at[pl.ds(sid * RS, RS)],
                    out_h.at[cid * NS + sid])


def _keys_body(src_ref, dst_ref, et_ref, key_ref, pair_ref, NN, R):
    et = et_ref[...]
    key_ref[...] = et * NN + src_ref[...]
    pair_ref[...] = dst_ref[...] * R + et


def _keys_call(src, dst, et, NN, R):
    E = src.shape[0]
    rows = E // 128
    body = functools.partial(_keys_body, NN=NN, R=R)
    key, pair = pl.pallas_call(
        body,
        out_shape=[
            jax.ShapeDtypeStruct((rows, 128), jnp.int32),
            jax.ShapeDtypeStruct((rows, 128), jnp.int32),
        ],
    )(src.reshape(rows, 128), dst.reshape(rows, 128), et.reshape(rows, 128))
    return key.reshape(E), pair.reshape(E)


def _norm_body(cnt_ref, norm_ref):
    c = cnt_ref[0] + cnt_ref[1]
    norm_ref[...] = 1.0 / jnp.maximum(c, 1.0)


def _norm_call(cnt_part, NP):
    rows = NP // 128
    cnt3 = cnt_part.reshape(NC, rows, 128)
    norm = pl.pallas_call(
        _norm_body,
        out_shape=jax.ShapeDtypeStruct((rows, 128), jnp.float32),
    )(cnt3)
    return norm.reshape(NP)


def _xform_body(x_ref, w_ref, root_ref, b_ref, h_ref, y_ref):
    xb = x_ref[...]
    for r in range(w_ref.shape[0]):
        h_ref[r] = jnp.dot(xb, w_ref[r], preferred_element_type=jnp.float32)
    y_ref[...] = (jnp.dot(xb, root_ref[...], preferred_element_type=jnp.float32)
                  + b_ref[...])


def _xform_call(x, W, root, b, BN=1000):
    NN, D = x.shape
    R = W.shape[0]
    grid = NN // BN
    h, y = pl.pallas_call(
        _xform_body,
        grid=(grid,),
        in_specs=[
            pl.BlockSpec((BN, D), lambda i: (i, 0)),
            pl.BlockSpec((R, D, D), lambda i: (0, 0, 0)),
            pl.BlockSpec((D, D), lambda i: (0, 0)),
            pl.BlockSpec((1, D), lambda i: (0, 0)),
        ],
        out_specs=[
            pl.BlockSpec((R, BN, D), lambda i: (0, i, 0)),
            pl.BlockSpec((BN, D), lambda i: (i, 0)),
        ],
        out_shape=[
            jax.ShapeDtypeStruct((R, NN, D), jnp.float32),
            jax.ShapeDtypeStruct((NN, D), jnp.float32),
        ],
    )(x, W, root, b.reshape(1, D))
    return h.reshape(R * NN, D), y


def _combine_xform_body(p_ref, yr_ref, w_ref, root_ref, b_ref, h_ref, y_ref):
    xb = jnp.maximum(p_ref[0] + p_ref[1] + yr_ref[...], 0.0)
    for r in range(w_ref.shape[0]):
        h_ref[r] = jnp.dot(xb, w_ref[r], preferred_element_type=jnp.float32)
    y_ref[...] = (jnp.dot(xb, root_ref[...], preferred_element_type=jnp.float32)
                  + b_ref[...])


def _combine_xform_call(p, yr, W, root, b, BN=1000):
    NN, D = yr.shape
    R = W.shape[0]
    grid = NN // BN
    p3 = p.reshape(NC, NN, D)
    h, y = pl.pallas_call(
        _combine_xform_body,
        grid=(grid,),
        in_specs=[
            pl.BlockSpec((NC, BN, D), lambda i: (0, i, 0)),
            pl.BlockSpec((BN, D), lambda i: (i, 0)),
            pl.BlockSpec((R, D, D), lambda i: (0, 0, 0)),
            pl.BlockSpec((D, D), lambda i: (0, 0)),
            pl.BlockSpec((1, D), lambda i: (0, 0)),
        ],
        out_specs=[
            pl.BlockSpec((R, BN, D), lambda i: (0, i, 0)),
            pl.BlockSpec((BN, D), lambda i: (i, 0)),
        ],
        out_shape=[
            jax.ShapeDtypeStruct((R, NN, D), jnp.float32),
            jax.ShapeDtypeStruct((NN, D), jnp.float32),
        ],
    )(p3, yr, W, root, b.reshape(1, D))
    return h.reshape(R * NN, D), y


def _final_body(p_ref, yr_ref, out_ref):
    out_ref[...] = p_ref[0] + p_ref[1] + yr_ref[...]


def _final_call(p, yr, BN=1000):
    NN, D = yr.shape
    grid = NN // BN
    p3 = p.reshape(NC, NN, D)
    return pl.pallas_call(
        _final_body,
        grid=(grid,),
        in_specs=[
            pl.BlockSpec((NC, BN, D), lambda i: (0, i, 0)),
            pl.BlockSpec((BN, D), lambda i: (i, 0)),
        ],
        out_specs=pl.BlockSpec((BN, D), lambda i: (i, 0)),
        out_shape=jax.ShapeDtypeStruct((NN, D), jnp.float32),
    )(p3, yr)


def kernel(x, edge_index, edge_type, W1, root1, b1, W2, root2, b2):
    NN, D = x.shape
    R = W1.shape[0]
    E = edge_type.shape[0]
    NP = NN * R
    NP2 = ((NP + NS * 128 - 1) // (NS * 128)) * (NS * 128)
    RS2 = ((NN // NS) + 7) // 8 * 8
    NN2 = NS * RS2

    key_flat, pair_flat = _keys_call(edge_index[0], edge_index[1],
                                     edge_type, NN, R)
    NCHT = E // (CR * B)
    key2 = key_flat.reshape(NCHT, CR, B)
    pair2 = pair_flat.reshape(NCHT, CR, B)
    dst2 = edge_index[1].reshape(NCHT, CR, B)
    zeros_cnt = jnp.zeros((NP2 // NS,), jnp.float32)
    zeros_row = jnp.zeros((RS2, D), jnp.float32)

    cnt_part = _make_counts_kernel(E, NP2)(pair2, zeros_cnt)
    norm = _norm_call(cnt_part, NP2)

    agg1 = _make_agg_kernel(E, D, RS2, True)
    agg2 = _make_agg_kernel(E, D, RS2, False)

    h1, yr1 = _xform_call(x, W1, root1, b1)
    p1, esc2 = agg1(h1, key2, dst2, pair2, norm, zeros_row)
    p1 = p1.reshape(NC, NN2, D)[:, :NN]
    h2, yr2 = _combine_xform_call(p1, yr1, W2, root2, b2)
    p2 = agg2(h2, key2, dst2, esc2, zeros_row)
    p2 = p2.reshape(NC, NN2, D)[:, :NN]
    return _final_call(p2, yr2)


# speedup vs baseline: 1.1424x; 1.1424x over previous
import functools

import jax
import jax.numpy as jnp
from jax import lax
from jax.experimental import pallas as pl
from jax.experimental.pallas import tpu as pltpu
from jax.experimental.pallas import tpu_sc as plsc

NC = 2
NS = 16
NW = NC * NS
LANES = 16
B = 80
CR = 25


def _make_counts_kernel(E, NP):
    EW = E // NW
    RW = EW // B
    NCH = RW // CR
    ST = NP // NS
    mesh = plsc.VectorSubcoreMesh(core_axis_name="c", subcore_axis_name="s")

    @functools.partial(
        pl.kernel,
        out_type=jax.ShapeDtypeStruct((NC * NP,), jnp.float32),
        mesh=mesh,
        compiler_params=pltpu.CompilerParams(needs_layout_passes=False),
        scratch_types=[
            pltpu.VMEM((CR, B), jnp.int32),
            pltpu.VMEM((B,), jnp.float32),
            pltpu.VMEM_SHARED((NP,), jnp.float32),
        ],
    )
    def counts_kernel(pair2_h, zeros_h, cnt_h, pairc, ones_v, cnt_sp):
        cid = lax.axis_index("c")
        sid = lax.axis_index("s")
        wid = sid * NC + cid
        for k in range(B // LANES):
            ones_v[pl.ds(k * LANES, LANES)] = jnp.ones((LANES,), jnp.float32)
        pltpu.sync_copy(zeros_h, cnt_sp.at[pl.ds(sid * ST, ST)])
        plsc.subcore_barrier()

        def chunk_body(c, carry):
            ch = wid * NCH + c
            pltpu.sync_copy(pair2_h.at[ch], pairc)

            def sb_body(sb, carry2):
                pltpu.sync_copy(ones_v, cnt_sp.at[pairc.at[sb]], add=True)
                return carry2

            return lax.fori_loop(0, CR, sb_body, carry)

        lax.fori_loop(0, NCH, chunk_body, 0)
        plsc.subcore_barrier()
        pltpu.sync_copy(cnt_sp.at[pl.ds(sid * ST, ST)],
                        cnt_h.at[pl.ds(cid * NP + sid * ST, ST)])

    return counts_kernel


def _make_agg_kernel(E, D, RS, first):
    EW = E // NW
    RW = EW // B
    NCH = RW // CR
    mesh = plsc.VectorSubcoreMesh(core_axis_name="c", subcore_axis_name="s")

    acc_t = jax.ShapeDtypeStruct((NC * NS, RS, D), jnp.float32)
    esc_t = jax.ShapeDtypeStruct((E // (CR * B), CR, B), jnp.float32)
    out_type = [acc_t, esc_t] if first else acc_t

    scratch = [
        pltpu.VMEM((CR, B), jnp.int32),
        pltpu.VMEM((CR, B), jnp.int32),
        pltpu.VMEM((CR, B), jnp.float32),
        pltpu.VMEM((B, D), jnp.float32),
        pltpu.VMEM((B, D), jnp.float32),
        pltpu.VMEM((B, D), jnp.float32),
        pltpu.VMEM_SHARED((NS * RS, D), jnp.float32),
    ] + [pltpu.SemaphoreType.DMA] * 9
    if first:
        scratch.insert(2, pltpu.VMEM((CR, B), jnp.int32))

    if first:
        @functools.partial(
            pl.kernel, out_type=out_type, mesh=mesh,
            compiler_params=pltpu.CompilerParams(needs_layout_passes=False),
            scratch_types=scratch,
        )
        def agg_kernel(h_h, key2_h, dst2_h, pair2_h, norm_h, zrow_h, *rest):
            _agg_body(h_h, key2_h, dst2_h, pair2_h, norm_h, zrow_h, rest,
                      E, D, RS, True)
    else:
        @functools.partial(
            pl.kernel, out_type=out_type, mesh=mesh,
            compiler_params=pltpu.CompilerParams(needs_layout_passes=False),
            scratch_types=scratch,
        )
        def agg_kernel(h_h, key2_h, dst2_h, esc2_h, zrow_h, *rest):
            _agg_body(h_h, key2_h, dst2_h, esc2_h, None, zrow_h, rest,
                      E, D, RS, False)

    return agg_kernel


def _agg_body(h_h, key2_h, dst2_h, aux2_h, norm_h, zrow_h, rest,
              E, D, RS, first):
    EW = E // NW
    RW = EW // B
    NCH = RW // CR
    if first:
        (out_h, esc_h, keyc, dstc, pairc, sclc,
         rows0, rows1, rows2, acc_sp,
         semr0, semr1, semr2, sems0, sems1, sems2,
         semw0, semw1, semw2) = rest
    else:
        (out_h, keyc, dstc, sclc,
         rows0, rows1, rows2, acc_sp,
         semr0, semr1, semr2, sems0, sems1, sems2,
         semw0, semw1, semw2) = rest
        pairc = None
        esc_h = None
    cid = lax.axis_index("c")
    sid = lax.axis_index("s")
    wid = sid * NC + cid
    pltpu.sync_copy(zrow_h, acc_sp.at[pl.ds(sid * RS, RS)])
    plsc.subcore_barrier()

    bufs = ((rows0, semr0, sems0, semw0),
            (rows1, semr1, sems1, semw1),
            (rows2, semr2, sems2, semw2))

    def issue(sb, p):
        rows, semr, sems, _ = bufs[p]
        pltpu.async_copy(h_h.at[keyc.at[sb]], rows, semr)
        if first:
            pltpu.async_copy(norm_h.at[pairc.at[sb]], sclc.at[sb], sems)

    def drain(sb, p):
        rows, semr, sems, _ = bufs[p]
        pltpu.make_async_copy(h_h.at[keyc.at[sb]], rows, semr).wait()
        if first:
            pltpu.make_async_copy(
                norm_h.at[pairc.at[sb]], sclc.at[sb], sems).wait()

    def scale(sb, p):
        rows = bufs[p][0]

        def row_body(q, carry3):
            for u in range(4):
                r = q * 4 + u
                sc16 = plsc.load_gather(
                    sclc.at[sb], [lax.broadcast(r, (LANES,))])
                for k in range(D // LANES):
                    sl = pl.ds(k * LANES, LANES)
                    rows[r, sl] = rows[r, sl] * sc16
            return carry3

        lax.fori_loop(0, B // 4, row_body, 0)

    def scat(sb, p):
        rows, _, _, semw = bufs[p]
        pltpu.async_copy(rows, acc_sp.at[dstc.at[sb]], semw, add=True)

    def wait_scat(sb, p):
        rows, _, _, semw = bufs[p]
        pltpu.make_async_copy(rows, acc_sp.at[dstc.at[sb]], semw).wait()

    def step(r, p, do_wait=True, do_issue=True):
        drain(r, p)
        scale(r, p)
        scat(r, p)
        if do_wait:
            wait_scat(r - 1, (p + 2) % 3)
        if do_issue:
            issue(r + 2, (p + 2) % 3)

    def chunk_body(c, carry):
        ch = wid * NCH + c
        pltpu.sync_copy(key2_h.at[ch], keyc)
        pltpu.sync_copy(dst2_h.at[ch], dstc)
        if first:
            pltpu.sync_copy(aux2_h.at[ch], pairc)
        else:
            pltpu.sync_copy(aux2_h.at[ch], sclc)

        issue(0, 0)
        issue(1, 1)
        step(0, 0, do_wait=False)

        def trio_body(j, carry2):
            r = 3 * j + 1
            step(r, 1)
            step(r + 1, 2)
            step(r + 2, 0)
            return carry2

        lax.fori_loop(0, (CR - 4) // 3, trio_body, 0)
        step(CR - 3, 1)
        step(CR - 2, 2, do_issue=False)
        step(CR - 1, 0, do_issue=False)
        wait_scat(CR - 1, 0)
        if first:
            pltpu.sync_copy(sclc, esc_h.at[ch])
        return carry

    lax.fori_loop(0, NCH, chunk_body, 0)
    plsc.subcore_barrier()
    pltpu.sync_copy(acc_sp.at[pl.ds(sid * RS, RS)],
                    out_h.at[cid * NS + sid])


def _keys_body(src_ref, dst_ref, et_ref, key_ref, pair_ref, NN, R):
    et = et_ref[...]
    key_ref[...] = et * NN + src_ref[...]
    pair_ref[...] = dst_ref[...] * R + et


def _keys_call(src, dst, et, NN, R):
    E = src.shape[0]
    rows = E // 128
    body = functools.partial(_keys_body, NN=NN, R=R)
    key, pair = pl.pallas_call(
        body,
        out_shape=[
            jax.ShapeDtypeStruct((rows, 128), jnp.int32),
            jax.ShapeDtypeStruct((rows, 128), jnp.int32),
        ],
    )(src.reshape(rows, 128), dst.reshape(rows, 128), et.reshape(rows, 128))
    return key.reshape(E), pair.reshape(E)


def _norm_body(cnt_ref, norm_ref):
    c = cnt_ref[0] + cnt_ref[1]
    norm_ref[...] = 1.0 / jnp.maximum(c, 1.0)


def _norm_call(cnt_part, NP):
    rows = NP // 128
    cnt3 = cnt_part.reshape(NC, rows, 128)
    norm = pl.pallas_call(
        _norm_body,
        out_shape=jax.ShapeDtypeStruct((rows, 128), jnp.float32),
    )(cnt3)
    return norm.reshape(NP)


def _xform_body(x_ref, w_ref, root_ref, b_ref, h_ref, y_ref):
    xb = x_ref[...]
    for r in range(w_ref.shape[0]):
        h_ref[r] = jnp.dot(xb, w_ref[r], preferred_element_type=jnp.float32)
    y_ref[...] = (jnp.dot(xb, root_ref[...], preferred_element_type=jnp.float32)
                  + b_ref[...])


def _xform_call(x, W, root, b, BN=1000):
    NN, D = x.shape
    R = W.shape[0]
    grid = NN // BN
    h, y = pl.pallas_call(
        _xform_body,
        grid=(grid,),
        in_specs=[
            pl.BlockSpec((BN, D), lambda i: (i, 0)),
            pl.BlockSpec((R, D, D), lambda i: (0, 0, 0)),
            pl.BlockSpec((D, D), lambda i: (0, 0)),
            pl.BlockSpec((1, D), lambda i: (0, 0)),
        ],
        out_specs=[
            pl.BlockSpec((R, BN, D), lambda i: (0, i, 0)),
            pl.BlockSpec((BN, D), lambda i: (i, 0)),
        ],
        out_shape=[
            jax.ShapeDtypeStruct((R, NN, D), jnp.float32),
            jax.ShapeDtypeStruct((NN, D), jnp.float32),
        ],
    )(x, W, root, b.reshape(1, D))
    return h.reshape(R * NN, D), y


def _combine_xform_body(p_ref, yr_ref, w_ref, root_ref, b_ref, h_ref, y_ref):
    xb = jnp.maximum(p_ref[0] + p_ref[1] + yr_ref[...], 0.0)
    for r in range(w_ref.shape[0]):
        h_ref[r] = jnp.dot(xb, w_ref[r], preferred_element_type=jnp.float32)
    y_ref[...] = (jnp.dot(xb, root_ref[...], preferred_element_type=jnp.float32)
                  + b_ref[...])


def _combine_xform_call(p, yr, W, root, b, BN=1000):
    NN, D = yr.shape
    R = W.shape[0]
    grid = NN // BN
    p3 = p.reshape(NC, NN, D)
    h, y = pl.pallas_call(
        _combine_xform_body,
        grid=(grid,),
        in_specs=[
            pl.BlockSpec((NC, BN, D), lambda i: (0, i, 0)),
            pl.BlockSpec((BN, D), lambda i: (i, 0)),
            pl.BlockSpec((R, D, D), lambda i: (0, 0, 0)),
            pl.BlockSpec((D, D), lambda i: (0, 0)),
            pl.BlockSpec((1, D), lambda i: (0, 0)),
        ],
        out_specs=[
            pl.BlockSpec((R, BN, D), lambda i: (0, i, 0)),
            pl.BlockSpec((BN, D), lambda i: (i, 0)),
        ],
        out_shape=[
            jax.ShapeDtypeStruct((R, NN, D), jnp.float32),
            jax.ShapeDtypeStruct((NN, D), jnp.float32),
        ],
    )(p3, yr, W, root, b.reshape(1, D))
    return h.reshape(R * NN, D), y


def _final_body(p_ref, yr_ref, out_ref):
    out_ref[...] = p_ref[0] + p_ref[1] + yr_ref[...]


def _final_call(p, yr, BN=1000):
    NN, D = yr.shape
    grid = NN // BN
    p3 = p.reshape(NC, NN, D)
    return pl.pallas_call(
        _final_body,
        grid=(grid,),
        in_specs=[
            pl.BlockSpec((NC, BN, D), lambda i: (0, i, 0)),
            pl.BlockSpec((BN, D), lambda i: (i, 0)),
        ],
        out_specs=pl.BlockSpec((BN, D), lambda i: (i, 0)),
        out_shape=jax.ShapeDtypeStruct((NN, D), jnp.float32),
    )(p3, yr)


def kernel(x, edge_index, edge_type, W1, root1, b1, W2, root2, b2):
    NN, D = x.shape
    R = W1.shape[0]
    E = edge_type.shape[0]
    NP = NN * R
    NP2 = ((NP + NS * 128 - 1) // (NS * 128)) * (NS * 128)
    RS2 = ((NN // NS) + 7) // 8 * 8
    NN2 = NS * RS2

    key_flat, pair_flat = _keys_call(edge_index[0], edge_index[1],
                                     edge_type, NN, R)
    NCHT = E // (CR * B)
    key2 = key_flat.reshape(NCHT, CR, B)
    pair2 = pair_flat.reshape(NCHT, CR, B)
    dst2 = edge_index[1].reshape(NCHT, CR, B)
    zeros_cnt = jnp.zeros((NP2 // NS,), jnp.float32)
    zeros_row = jnp.zeros((RS2, D), jnp.float32)

    cnt_part = _make_counts_kernel(E, NP2)(pair2, zeros_cnt)
    norm = _norm_call(cnt_part, NP2)

    agg1 = _make_agg_kernel(E, D, RS2, True)
    agg2 = _make_agg_kernel(E, D, RS2, False)

    h1, yr1 = _xform_call(x, W1, root1, b1)
    p1, esc2 = agg1(h1, key2, dst2, pair2, norm, zeros_row)
    p1 = p1.reshape(NC, NN2, D)[:, :NN]
    h2, yr2 = _combine_xform_call(p1, yr1, W2, root2, b2)
    p2 = agg2(h2, key2, dst2, esc2, zeros_row)
    p2 = p2.reshape(NC, NN2, D)[:, :NN]
    return _final_call(p2, yr2)
